# baseline (speedup 1.0000x reference)
import functools

import jax
import jax.numpy as jnp
from jax import lax
from jax.experimental import pallas as pl
from jax.experimental.pallas import tpu as pltpu, tpu_sc as plsc

NUM_TOKENS = 1000000
DIMS = 64
BATCH = 4
SEQ = 2048

NC, NS = 2, 16
NW = NC * NS
B = BATCH * SEQ
B_PER_W = B // NW
CHUNK = 128
N_CHUNK = B_PER_W // CHUNK

_mesh = plsc.VectorSubcoreMesh(core_axis_name="c", subcore_axis_name="s")


@functools.partial(
    pl.kernel,
    mesh=_mesh,
    out_type=jax.ShapeDtypeStruct((B, DIMS), jnp.float32),
    scratch_types=[
        pltpu.VMEM((N_CHUNK, CHUNK), jnp.int32),
        pltpu.VMEM((B_PER_W, DIMS), jnp.float32),
        pltpu.SemaphoreType.DMA,
    ],
    compiler_params=pltpu.CompilerParams(use_tc_tiling_on_sc=False),
)
def _gather_kernel(idx_hbm, table_hbm, out_hbm, idx_v, rows_v, sem):
    wid = lax.axis_index("s") * NC + lax.axis_index("c")
    base = wid * B_PER_W
    pltpu.sync_copy(idx_hbm.at[wid], idx_v)
    copies = []
    for j in range(N_CHUNK):
        copies.append(
            pltpu.async_copy(
                table_hbm.at[idx_v.at[j]],
                rows_v.at[pl.ds(j * CHUNK, CHUNK)],
                sem,
            )
        )
    for c in copies:
        c.wait()
    pltpu.sync_copy(rows_v, out_hbm.at[pl.ds(base, B_PER_W)])


def kernel(tokens, table):
    idx = jnp.reshape(tokens.astype(jnp.int32), (NW, N_CHUNK, CHUNK))
    out = _gather_kernel(idx, table)
    return jnp.reshape(out, (BATCH, SEQ, DIMS))


# speedup vs baseline: 2.2433x; 2.2433x over previous
import functools

import jax
import jax.numpy as jnp
from jax import lax
from jax.experimental import pallas as pl
from jax.experimental.pallas import tpu as pltpu, tpu_sc as plsc

NUM_TOKENS = 1000000
DIMS = 64
BATCH = 4
SEQ = 2048

NC, NS = 2, 16
NW = NC * NS
B = BATCH * SEQ
B_PER_W = B // NW
LANES = 16
BLK = 128

_mesh = plsc.VectorSubcoreMesh(core_axis_name="c", subcore_axis_name="s")


@functools.partial(
    pl.kernel,
    mesh=_mesh,
    out_type=jax.ShapeDtypeStruct((BATCH, DIMS, SEQ), jnp.float32),
    scratch_types=[
        pltpu.VMEM((B_PER_W,), jnp.int32),
        pltpu.VMEM((DIMS, BLK), jnp.float32),
        pltpu.VMEM((DIMS, BLK), jnp.float32),
        pltpu.VMEM((DIMS, B_PER_W), jnp.float32),
        pltpu.SemaphoreType.DMA,
    ],
    compiler_params=pltpu.CompilerParams(needs_layout_passes=False),
)
def _blkgather(idx_hbm, tableT_hbm, out_hbm, idx_v, blk0, blk1, cols_v, sem):
    wid = lax.axis_index("s") * NC + lax.axis_index("c")
    base = wid * B_PER_W
    b = base // SEQ
    s0 = base % SEQ
    pltpu.sync_copy(idx_hbm.at[pl.ds(base, B_PER_W)], idx_v)

    bufs = (blk0, blk1)
    d16 = lax.iota(jnp.int32, LANES)

    def fire(row, buf):
        c0 = pl.multiple_of((row // BLK) * BLK, BLK)
        pltpu.async_copy(tableT_hbm.at[:, pl.ds(c0, BLK)], buf, sem)

    def drain(buf):
        pltpu.make_async_copy(tableT_hbm.at[:, pl.ds(0, BLK)], buf, sem).wait()

    def extract(row, i, buf):
        col = jnp.full((LANES,), row % BLK, jnp.int32)
        pos = jnp.full((LANES,), i, jnp.int32)
        for k in range(DIMS // LANES):
            dk = d16 + (k * LANES)
            vals = plsc.load_gather(buf, [dk, col])
            plsc.store_scatter(cols_v, [dk, pos], vals)

    def body(g, _):
        v = idx_v[pl.ds(g * LANES, LANES)]
        fire(v[0], bufs[0])
        for e in range(LANES):
            i = g * LANES + e
            drain(bufs[e % 2])
            if e + 1 < LANES:
                fire(v[e + 1], bufs[(e + 1) % 2])
            extract(v[e], i, bufs[e % 2])
        return ()

    lax.fori_loop(0, B_PER_W // LANES, body, ())
    pltpu.sync_copy(cols_v, out_hbm.at[b, :, pl.ds(s0, B_PER_W)])


def kernel(tokens, table):
    idx = jnp.reshape(tokens.astype(jnp.int32), (B,))
    out3 = _blkgather(idx, table.T)
    return jnp.transpose(out3, (0, 2, 1))
